# baseline (speedup 1.0000x reference)
import functools

import jax
import jax.numpy as jnp
from jax import lax
from jax.experimental import pallas as pl
from jax.experimental.pallas import tpu as pltpu
from jax.experimental.pallas import tpu_sc as plsc

VOCAB = 100000
DIM = 2048
TOTAL = 4 * 4096

NC, NS = 2, 16
NW = NC * NS
BPW = TOTAL // NW
CHUNK = 16
NBUF = 2
NCH = BPW // CHUNK
NGRP = NCH // NBUF


def _emb_kernel(ids_hbm, table_hbm, out_hbm, idx_v, bufs, gs0, gs1, ss0, ss1):
    gsems = [gs0, gs1]
    ssems = [ss0, ss1]
    wid = lax.axis_index("s") * NC + lax.axis_index("c")
    base = wid * BPW

    pltpu.sync_copy(ids_hbm.at[pl.ds(base, BPW)], idx_v)

    def start_gather(j, b):
        off = pl.multiple_of(j * CHUNK, 8)
        pltpu.async_copy(
            table_hbm.at[idx_v.at[pl.ds(off, CHUNK)]], bufs.at[b], gsems[b])

    def wait_gather(b):
        pltpu.make_async_copy(
            table_hbm.at[pl.ds(0, CHUNK)], bufs.at[b], gsems[b]).wait()

    def start_scatter(j, b):
        row = pl.multiple_of(base + j * CHUNK, 8)
        pltpu.async_copy(bufs.at[b], out_hbm.at[pl.ds(row, CHUNK)], ssems[b])

    def wait_scatter(b):
        pltpu.make_async_copy(
            bufs.at[b], out_hbm.at[pl.ds(0, CHUNK)], ssems[b]).wait()

    for b in range(NBUF):
        start_gather(b, b)

    def group(g, carry):
        for b in range(NBUF):
            wait_gather(b)
            start_scatter(g * NBUF + b, b)
        for b in range(NBUF):
            wait_scatter(b)
            start_gather(g * NBUF + b + NBUF, b)
        return carry

    lax.fori_loop(0, NGRP - 1, group, 0)

    for b in range(NBUF):
        wait_gather(b)
        start_scatter((NGRP - 1) * NBUF + b, b)
    for b in range(NBUF):
        wait_scatter(b)


@jax.jit
def _lookup(ids_flat, table):
    mesh = plsc.VectorSubcoreMesh(core_axis_name="c", subcore_axis_name="s")
    fn = functools.partial(
        pl.kernel,
        out_type=jax.ShapeDtypeStruct((TOTAL, DIM), jnp.float32),
        mesh=mesh,
        scratch_types=[
            pltpu.VMEM((BPW,), jnp.int32),
            pltpu.VMEM((NBUF, CHUNK, DIM), jnp.float32),
            pltpu.SemaphoreType.DMA,
            pltpu.SemaphoreType.DMA,
            pltpu.SemaphoreType.DMA,
            pltpu.SemaphoreType.DMA,
        ],
    )(_emb_kernel)
    return fn(ids_flat, table)


def kernel(input_ids, token_embeddings):
    ids_flat = input_ids.reshape(-1).astype(jnp.int32)
    out = _lookup(ids_flat, token_embeddings)
    return out.reshape(input_ids.shape + (token_embeddings.shape[1],))


# speedup vs baseline: 1.0580x; 1.0580x over previous
import functools

import jax
import jax.numpy as jnp
from jax import lax
from jax.experimental import pallas as pl
from jax.experimental.pallas import tpu as pltpu
from jax.experimental.pallas import tpu_sc as plsc

VOCAB = 100000
DIM = 2048
TOTAL = 4 * 4096

NC, NS = 2, 16
NW = NC * NS
BPW = TOTAL // NW
CHUNK = 16
NBUF = 2
NCH = BPW // CHUNK
NGRP = NCH // NBUF


def _emb_kernel(ids_hbm, table_hbm, out_hbm, idx_v, bufs, gs0, gs1, ss0, ss1):
    gsems = [gs0, gs1]
    ssems = [ss0, ss1]
    wid = lax.axis_index("s") * NC + lax.axis_index("c")
    base = wid * BPW

    pltpu.sync_copy(ids_hbm.at[pl.ds(base, BPW)], idx_v)

    def start_gather(j, b):
        off = pl.multiple_of(j * CHUNK, 8)
        pltpu.async_copy(
            table_hbm.at[idx_v.at[pl.ds(off, CHUNK)]], bufs.at[b], gsems[b])

    def wait_gather(b):
        pltpu.make_async_copy(
            table_hbm.at[pl.ds(0, CHUNK)], bufs.at[b], gsems[b]).wait()

    def start_scatter(j, b):
        row = pl.multiple_of(base + j * CHUNK, 8)
        pltpu.async_copy(bufs.at[b], out_hbm.at[pl.ds(row, CHUNK)], ssems[b])

    def wait_scatter(b):
        pltpu.make_async_copy(
            bufs.at[b], out_hbm.at[pl.ds(0, CHUNK)], ssems[b]).wait()

    for b in range(NBUF):
        start_gather(b, b)

    def group(g, carry):
        for b in range(NBUF):
            j = g * NBUF + b
            wait_gather(b)
            start_scatter(j, b)
            wait_scatter(b)
            start_gather(j + NBUF, b)
        return carry

    lax.fori_loop(0, NGRP - 1, group, 0)

    for b in range(NBUF):
        j = (NGRP - 1) * NBUF + b
        wait_gather(b)
        start_scatter(j, b)
        wait_scatter(b)


@jax.jit
def _lookup(ids_flat, table):
    mesh = plsc.VectorSubcoreMesh(core_axis_name="c", subcore_axis_name="s")
    fn = functools.partial(
        pl.kernel,
        out_type=jax.ShapeDtypeStruct((TOTAL, DIM), jnp.float32),
        mesh=mesh,
        scratch_types=[
            pltpu.VMEM((BPW,), jnp.int32),
            pltpu.VMEM((NBUF, CHUNK, DIM), jnp.float32),
            pltpu.SemaphoreType.DMA,
            pltpu.SemaphoreType.DMA,
            pltpu.SemaphoreType.DMA,
            pltpu.SemaphoreType.DMA,
        ],
    )(_emb_kernel)
    return fn(ids_flat, table)


def kernel(input_ids, token_embeddings):
    ids_flat = input_ids.reshape(-1).astype(jnp.int32)
    out = _lookup(ids_flat, token_embeddings)
    return out.reshape(input_ids.shape + (token_embeddings.shape[1],))
